# baseline (speedup 1.0000x reference)
import functools

import jax
import jax.numpy as jnp
from jax import lax
from jax.experimental import pallas as pl
from jax.experimental.pallas import tpu as pltpu
from jax.experimental.pallas import tpu_sc as plsc

NUM_EXPERTS = 64
TOP_K = 2
BLOCK_M = 1024
NUM_TOKENS = 16384

_SC_INFO = plsc.get_sparse_core_info()
_NC, _NS, _L = _SC_INFO.num_cores, _SC_INFO.num_subcores, _SC_INFO.num_lanes
_NW = _NC * _NS
_TOK_PER_W = NUM_TOKENS // _NW
_GROUPS = _TOK_PER_W // _L

_MAGN = 0x7FFFFFFF


_INT_MIN = -2147483648
_EGROUPS = 4
_NCAND = 2 * _EGROUPS


def _matmul_block(x_ref, w_ref, logits_ref, keys_t_ref):
    l = jnp.dot(x_ref[...], w_ref[...], preferred_element_type=jnp.float32)
    logits_ref[...] = l
    b = lax.bitcast_convert_type(l, jnp.int32)
    s = b ^ (jnp.right_shift(b, 31) & _MAGN)
    e = lax.broadcasted_iota(jnp.int32, l.shape, 1)
    key = (s & ~63) | (63 - e)
    cols = []
    gw = NUM_EXPERTS // _EGROUPS
    for g in range(_EGROUPS):
        kg = key[:, g * gw:(g + 1) * gw]
        m1 = jnp.max(kg, axis=1, keepdims=True)
        m2 = jnp.max(jnp.where(kg == m1, _INT_MIN, kg), axis=1, keepdims=True)
        cols.append(m1)
        cols.append(m2)
    cand = jnp.concatenate(cols, axis=1)
    ct3 = cand.T.reshape(_NCAND, BLOCK_M // _TOK_PER_W, _TOK_PER_W)
    keys_t_ref[...] = jnp.swapaxes(ct3, 0, 1)


def _tc_matmul(x, W):
    n, d = x.shape
    num_e = W.shape[1]
    return pl.pallas_call(
        _matmul_block,
        grid=(n // BLOCK_M,),
        in_specs=[
            pl.BlockSpec((BLOCK_M, d), lambda i: (i, 0)),
            pl.BlockSpec((d, num_e), lambda i: (0, 0)),
        ],
        out_specs=[
            pl.BlockSpec((BLOCK_M, num_e), lambda i: (i, 0)),
            pl.BlockSpec((BLOCK_M // _TOK_PER_W, _NCAND, _TOK_PER_W),
                         lambda i: (i, 0, 0)),
        ],
        out_shape=[
            jax.ShapeDtypeStruct((n, num_e), jnp.float32),
            jax.ShapeDtypeStruct((_NW, _NCAND, _TOK_PER_W), jnp.int32),
        ],
        compiler_params=pltpu.CompilerParams(
            dimension_semantics=("parallel",),
        ),
    )(x, W)


def _sc_router(keys_t_hbm, ids1_hbm, ids2_hbm, p1_hbm, p2_hbm,
               kt_v, i1_v, i2_v, p1_v, p2_v):
    wid = lax.axis_index("s") * _NC + lax.axis_index("c")
    base = wid * _TOK_PER_W
    pltpu.sync_copy(keys_t_hbm.at[wid], kt_v)

    int_min = jnp.full((_L,), -2147483648, jnp.int32)

    def group_body(g, carry):
        m1, m2 = int_min, int_min
        col = g * _L
        for c in range(_NCAND):
            k = kt_v[c, pl.ds(col, _L)]
            gt = k > m1
            m2 = jnp.where(gt, m1, jnp.maximum(m2, k))
            m1 = jnp.maximum(m1, k)
        i1_v[pl.ds(col, _L)] = 63 - (m1 & 63)
        i2_v[pl.ds(col, _L)] = 63 - (m2 & 63)
        b1 = m1 ^ (jnp.right_shift(m1, 31) & _MAGN)
        b2 = m2 ^ (jnp.right_shift(m2, 31) & _MAGN)
        v1 = lax.bitcast_convert_type(b1, jnp.float32)
        v2 = lax.bitcast_convert_type(b2, jnp.float32)
        e2 = jnp.exp(v2 - v1)
        p1 = 1.0 / (1.0 + e2)
        p1_v[pl.ds(col, _L)] = p1
        p2_v[pl.ds(col, _L)] = 1.0 - p1
        return carry

    lax.fori_loop(0, _GROUPS, group_body, 0)

    pltpu.sync_copy(i1_v, ids1_hbm.at[pl.ds(base, _TOK_PER_W)])
    pltpu.sync_copy(i2_v, ids2_hbm.at[pl.ds(base, _TOK_PER_W)])
    pltpu.sync_copy(p1_v, p1_hbm.at[pl.ds(base, _TOK_PER_W)])
    pltpu.sync_copy(p2_v, p2_hbm.at[pl.ds(base, _TOK_PER_W)])


_sc_router_call = functools.partial(
    pl.kernel,
    mesh=plsc.VectorSubcoreMesh(core_axis_name="c", subcore_axis_name="s"),
    out_type=[
        jax.ShapeDtypeStruct((NUM_TOKENS,), jnp.int32),
        jax.ShapeDtypeStruct((NUM_TOKENS,), jnp.int32),
        jax.ShapeDtypeStruct((NUM_TOKENS,), jnp.float32),
        jax.ShapeDtypeStruct((NUM_TOKENS,), jnp.float32),
    ],
    scratch_types=[
        pltpu.VMEM((_NCAND, _TOK_PER_W), jnp.int32),
        pltpu.VMEM((_TOK_PER_W,), jnp.int32),
        pltpu.VMEM((_TOK_PER_W,), jnp.int32),
        pltpu.VMEM((_TOK_PER_W,), jnp.float32),
        pltpu.VMEM((_TOK_PER_W,), jnp.float32),
    ],
)(_sc_router)


@jax.jit
def kernel(x, W):
    if x.ndim == 3:
        x = x.reshape(-1, x.shape[-1])
    logits, keys_t = _tc_matmul(x, W)
    ids1, ids2, p1, p2 = _sc_router_call(keys_t)
    ids = jnp.stack([ids1, ids2], axis=-1)
    probs = jnp.stack([p1, p2], axis=-1)
    return ids, probs, logits


# speedup vs baseline: 1.0513x; 1.0513x over previous
import functools

import jax
import jax.numpy as jnp
from jax import lax
from jax.experimental import pallas as pl
from jax.experimental.pallas import tpu as pltpu
from jax.experimental.pallas import tpu_sc as plsc

NUM_EXPERTS = 64
TOP_K = 2
BLOCK_M = 1024
NUM_TOKENS = 16384

_SC_INFO = plsc.get_sparse_core_info()
_NC, _NS, _L = _SC_INFO.num_cores, _SC_INFO.num_subcores, _SC_INFO.num_lanes
_NW = _NC * _NS
_TOK_PER_W = NUM_TOKENS // _NW
_GROUPS = _TOK_PER_W // _L

_MAGN = 0x7FFFFFFF


_INT_MIN = -2147483648
_EGROUPS = 2
_NCAND = 2 * _EGROUPS


def _matmul_block(x_ref, w_ref, logits_ref, keys_t_ref):
    l = jnp.dot(x_ref[...], w_ref[...], preferred_element_type=jnp.float32)
    logits_ref[...] = l
    b = lax.bitcast_convert_type(l, jnp.int32)
    s = b ^ (jnp.right_shift(b, 31) & _MAGN)
    e = lax.broadcasted_iota(jnp.int32, l.shape, 1)
    key = (s & ~63) | (63 - e)
    cols = []
    gw = NUM_EXPERTS // _EGROUPS
    for g in range(_EGROUPS):
        kg = key[:, g * gw:(g + 1) * gw]
        m1 = jnp.max(kg, axis=1, keepdims=True)
        m2 = jnp.max(jnp.where(kg == m1, _INT_MIN, kg), axis=1, keepdims=True)
        cols.append(m1)
        cols.append(m2)
    cand = jnp.concatenate(cols, axis=1)
    ct3 = cand.T.reshape(_NCAND, BLOCK_M // _TOK_PER_W, _TOK_PER_W)
    keys_t_ref[...] = jnp.swapaxes(ct3, 0, 1)


def _tc_matmul(x, W):
    n, d = x.shape
    num_e = W.shape[1]
    return pl.pallas_call(
        _matmul_block,
        grid=(n // BLOCK_M,),
        in_specs=[
            pl.BlockSpec((BLOCK_M, d), lambda i: (i, 0)),
            pl.BlockSpec((d, num_e), lambda i: (0, 0)),
        ],
        out_specs=[
            pl.BlockSpec((BLOCK_M, num_e), lambda i: (i, 0)),
            pl.BlockSpec((BLOCK_M // _TOK_PER_W, _NCAND, _TOK_PER_W),
                         lambda i: (i, 0, 0)),
        ],
        out_shape=[
            jax.ShapeDtypeStruct((n, num_e), jnp.float32),
            jax.ShapeDtypeStruct((_NW, _NCAND, _TOK_PER_W), jnp.int32),
        ],
        compiler_params=pltpu.CompilerParams(
            dimension_semantics=("parallel",),
        ),
    )(x, W)


def _sc_router(keys_t_hbm, ids1_hbm, ids2_hbm, p1_hbm, p2_hbm,
               kt_v, i1_v, i2_v, p1_v, p2_v):
    wid = lax.axis_index("s") * _NC + lax.axis_index("c")
    base = wid * _TOK_PER_W
    pltpu.sync_copy(keys_t_hbm.at[wid], kt_v)

    int_min = jnp.full((_L,), -2147483648, jnp.int32)

    def group_body(g, carry):
        m1, m2 = int_min, int_min
        col = g * _L
        for c in range(_NCAND):
            k = kt_v[c, pl.ds(col, _L)]
            gt = k > m1
            m2 = jnp.where(gt, m1, jnp.maximum(m2, k))
            m1 = jnp.maximum(m1, k)
        i1_v[pl.ds(col, _L)] = 63 - (m1 & 63)
        i2_v[pl.ds(col, _L)] = 63 - (m2 & 63)
        b1 = m1 ^ (jnp.right_shift(m1, 31) & _MAGN)
        b2 = m2 ^ (jnp.right_shift(m2, 31) & _MAGN)
        v1 = lax.bitcast_convert_type(b1, jnp.float32)
        v2 = lax.bitcast_convert_type(b2, jnp.float32)
        e2 = jnp.exp(v2 - v1)
        p1 = 1.0 / (1.0 + e2)
        p1_v[pl.ds(col, _L)] = p1
        p2_v[pl.ds(col, _L)] = 1.0 - p1
        return carry

    lax.fori_loop(0, _GROUPS, group_body, 0)

    pltpu.sync_copy(i1_v, ids1_hbm.at[pl.ds(base, _TOK_PER_W)])
    pltpu.sync_copy(i2_v, ids2_hbm.at[pl.ds(base, _TOK_PER_W)])
    pltpu.sync_copy(p1_v, p1_hbm.at[pl.ds(base, _TOK_PER_W)])
    pltpu.sync_copy(p2_v, p2_hbm.at[pl.ds(base, _TOK_PER_W)])


_sc_router_call = functools.partial(
    pl.kernel,
    mesh=plsc.VectorSubcoreMesh(core_axis_name="c", subcore_axis_name="s"),
    out_type=[
        jax.ShapeDtypeStruct((NUM_TOKENS,), jnp.int32),
        jax.ShapeDtypeStruct((NUM_TOKENS,), jnp.int32),
        jax.ShapeDtypeStruct((NUM_TOKENS,), jnp.float32),
        jax.ShapeDtypeStruct((NUM_TOKENS,), jnp.float32),
    ],
    scratch_types=[
        pltpu.VMEM((_NCAND, _TOK_PER_W), jnp.int32),
        pltpu.VMEM((_TOK_PER_W,), jnp.int32),
        pltpu.VMEM((_TOK_PER_W,), jnp.int32),
        pltpu.VMEM((_TOK_PER_W,), jnp.float32),
        pltpu.VMEM((_TOK_PER_W,), jnp.float32),
    ],
)(_sc_router)


@jax.jit
def kernel(x, W):
    if x.ndim == 3:
        x = x.reshape(-1, x.shape[-1])
    logits, keys_t = _tc_matmul(x, W)
    ids1, ids2, p1, p2 = _sc_router_call(keys_t)
    ids = jnp.stack([ids1, ids2], axis=-1)
    probs = jnp.stack([p1, p2], axis=-1)
    return ids, probs, logits
